# baseline (speedup 1.0000x reference)
import functools

import jax
import jax.numpy as jnp
import numpy as np
from jax import lax
from jax.experimental import pallas as pl
from jax.experimental.pallas import tpu as pltpu
from jax.experimental.pallas import tpu_sc as plsc

N = 10000
E = 320000
D = 128

NC = 2
NS = 16
T = NC * NS

CH = 128
NCH = -(-E // (T * CH))
EPAD = T * NCH * CH

ECH = 48
ENCH = 212
NSLOT = 4
EEPAD = T * ENCH * ECH

NPAD = 10112
RPT = NPAD // NS

_l = np.arange(D)
DPERM = (_l // 32) * 32 + (_l % 32 % 2) * 16 + (_l % 32) // 2

_mesh = plsc.VectorSubcoreMesh(core_axis_name="c", subcore_axis_name="s")


@functools.partial(
    pl.kernel,
    out_type=jax.ShapeDtypeStruct((NC * NPAD,), jnp.float32),
    mesh=_mesh,
    scratch_types=[
        pltpu.VMEM((NCH, CH), jnp.int32),
        pltpu.VMEM((NCH, CH), jnp.float32),
        pltpu.VMEM((640,), jnp.float32),
        pltpu.VMEM_SHARED((NPAD,), jnp.float32),
    ],
)
def _deg_kernel(col_hbm, ew_hbm, out_hbm, col_v, ew_v, buf_v, deg_sh):
    c = lax.axis_index("c")
    s = lax.axis_index("s")
    wid = s * NC + c

    zero16 = jnp.zeros((16,), jnp.float32)
    for i in range(640 // 16):
        buf_v[pl.ds(i * 16, 16)] = zero16
    pltpu.sync_copy(buf_v.at[pl.ds(0, RPT)], deg_sh.at[pl.ds(s * RPT, RPT)])
    plsc.subcore_barrier()

    pltpu.sync_copy(col_hbm.at[wid], col_v)
    pltpu.sync_copy(ew_hbm.at[wid], ew_v)

    def chunk(ch, carry):
        pltpu.sync_copy(ew_v.at[ch], deg_sh.at[col_v.at[ch]], add=True)
        return carry

    lax.fori_loop(0, NCH, chunk, 0)
    plsc.subcore_barrier()

    pltpu.sync_copy(deg_sh.at[pl.ds(s * RPT, RPT)], buf_v.at[pl.ds(0, RPT)])
    pltpu.sync_copy(buf_v.at[pl.ds(0, RPT)],
                    out_hbm.at[pl.ds(c * NPAD + s * RPT, RPT)])


@functools.partial(
    pl.kernel,
    out_type=jax.ShapeDtypeStruct((NC, NPAD, D), jnp.float32),
    mesh=_mesh,
    scratch_types=[
        pltpu.VMEM((NSLOT, 2, ECH), jnp.int32),
        pltpu.VMEM((NSLOT, ECH, 16), jnp.float32),
        pltpu.VMEM((NSLOT, ECH, D // 2), jnp.int32),
        pltpu.VMEM((ECH, D), jnp.float32),
        pltpu.VMEM_SHARED((NPAD, D), jnp.float32),
        pltpu.SemaphoreType.DMA,
        pltpu.SemaphoreType.DMA,
        pltpu.SemaphoreType.DMA,
        pltpu.SemaphoreType.DMA,
        pltpu.SemaphoreType.DMA,
        pltpu.SemaphoreType.DMA,
        pltpu.SemaphoreType.DMA,
        pltpu.SemaphoreType.DMA,
    ],
    compiler_params=pltpu.CompilerParams(use_tc_tiling_on_sc=False,
                                         needs_layout_passes=False),
)
def _edge_kernel(rc_hbm, ew_hbm, g_hbm, out_hbm,
                 rc_v, ew_v, rbuf, obuf, acc_sh,
                 gsem0, gsem1, gsem2, gsem3, psem0, psem1, psem2, psem3):
    c = lax.axis_index("c")
    s = lax.axis_index("s")
    wid = s * NC + c
    gsem = (gsem0, gsem1, gsem2, gsem3)
    psem = (psem0, psem1, psem2, psem3)

    zero16 = jnp.zeros((16,), jnp.float32)

    def zrow(r, carry):
        for j in range(D // 16):
            obuf[r, pl.ds(j * 16, 16)] = zero16
        return carry

    lax.fori_loop(0, ECH, zrow, 0)
    for i in range(RPT // ECH):
        pltpu.sync_copy(obuf, acc_sh.at[pl.ds(s * RPT + i * ECH, ECH)])
    rem = RPT - (RPT // ECH) * ECH
    if rem:
        pltpu.sync_copy(obuf.at[pl.ds(0, rem)],
                        acc_sh.at[pl.ds(s * RPT + (RPT // ECH) * ECH, rem)])
    plsc.subcore_barrier()

    def start_pref(ch, slot):
        pltpu.async_copy(rc_hbm.at[wid, ch], rc_v.at[slot], psem[slot])
        pltpu.async_copy(ew_hbm.at[wid, ch], ew_v.at[slot], psem[slot])

    def wait_pref(slot):
        pltpu.make_async_copy(rc_hbm.at[0, 0], rc_v.at[slot], psem[slot]).wait()
        pltpu.make_async_copy(ew_hbm.at[0, 0], ew_v.at[slot], psem[slot]).wait()

    def start_gather(slot):
        pltpu.async_copy(g_hbm.at[rc_v.at[slot, 0]], rbuf.at[slot], gsem[slot])

    def wait_gather(slot):
        pltpu.make_async_copy(g_hbm.at[rc_v.at[slot, 0]], rbuf.at[slot],
                              gsem[slot]).wait()

    def scale_scatter(slot):
        def edge(e, ecarry):
            ews = ew_v[slot, e]
            for k in range(D // 32):
                v = rbuf[slot, e, pl.ds(k * 16, 16)]
                lo = plsc.bitcast(v << 16, jnp.float32)
                hi = plsc.bitcast(v & jnp.int32(-65536), jnp.float32)
                obuf[e, pl.ds(k * 32, 16)] = lo * ews
                obuf[e, pl.ds(k * 32 + 16, 16)] = hi * ews
            return ecarry

        lax.fori_loop(0, ECH, edge, 0)
        pltpu.sync_copy(obuf, acc_sh.at[rc_v.at[slot, 1]], add=True)

    last = ENCH - 1

    for k in range(NSLOT - 1):
        start_pref(k, k)
        wait_pref(k)
        start_gather(k)
    start_pref(NSLOT - 1, NSLOT - 1)

    def ring(g, carry):
        base = NSLOT * g
        for k in range(NSLOT):
            km1 = (k + NSLOT - 1) % NSLOT
            wait_gather(k)
            wait_pref(km1)
            start_gather(km1)
            scale_scatter(k)
            start_pref(jnp.minimum(base + k + NSLOT, last), k)
        return carry

    lax.fori_loop(0, ENCH // NSLOT, ring, 0)
    for k in range(NSLOT - 1):
        wait_gather(k)
    wait_pref(NSLOT - 1)
    plsc.subcore_barrier()

    for i in range(RPT // ECH):
        pltpu.sync_copy(acc_sh.at[pl.ds(s * RPT + i * ECH, ECH)], obuf)
        pltpu.sync_copy(obuf, out_hbm.at[c, pl.ds(s * RPT + i * ECH, ECH)])
    if rem:
        pltpu.sync_copy(acc_sh.at[pl.ds(s * RPT + (RPT // ECH) * ECH, rem)],
                        obuf.at[pl.ds(0, rem)])
        pltpu.sync_copy(obuf.at[pl.ds(0, rem)],
                        out_hbm.at[c, pl.ds(s * RPT + (RPT // ECH) * ECH, rem)])


_BN = 1000
_GRID = N // _BN

_row_spec = pl.BlockSpec((_BN, D), lambda i: (i, 0))
_col_spec = pl.BlockSpec((_BN, 1), lambda i: (i, 0))
_mat_spec = pl.BlockSpec((D, D), lambda i: (0, 0))
_vec_spec = pl.BlockSpec((1, D), lambda i: (0, 0))


def _gelu(v):
    return 0.5 * v * (1.0 + lax.erf(v * 0.7071067811865476))


def _layernorm(v, w, b):
    m = jnp.mean(v, axis=-1, keepdims=True)
    var = jnp.mean((v - m) ** 2, axis=-1, keepdims=True)
    return (v - m) * lax.rsqrt(var + 1e-5) * w + b


def _tc_a_body(deg0, deg1, x, w1, g1_out, dinv_out):
    dinv = lax.rsqrt(deg0[...] + deg1[...] + 1.0)
    h = jnp.dot(x[...], w1[...], preferred_element_type=jnp.float32)
    g1_out[...] = dinv * h
    dinv_out[...] = dinv


def _tc_a(deg0, deg1, x, w1):
    return pl.pallas_call(
        _tc_a_body,
        grid=(_GRID,),
        in_specs=[_col_spec, _col_spec, _row_spec, _mat_spec],
        out_specs=[_row_spec, _col_spec],
        out_shape=[
            jax.ShapeDtypeStruct((N, D), jnp.float32),
            jax.ShapeDtypeStruct((N, 1), jnp.float32),
        ],
    )(deg0, deg1, x, w1)


def _tc_b_body(s0, s1, g1, dinv, b1, lnw, lnb, w2, g2_out):
    dv = dinv[...]
    v = dv * (s0[...] + s1[...] + g1[...]) + b1[...]
    v = _gelu(_layernorm(v, lnw[...], lnb[...]))
    g2_out[...] = dv * jnp.dot(v, w2[...], preferred_element_type=jnp.float32)


def _tc_b(s0, s1, g1, dinv, b1, lnw, lnb, w2):
    return pl.pallas_call(
        _tc_b_body,
        grid=(_GRID,),
        in_specs=[_row_spec, _row_spec, _row_spec, _col_spec,
                  _vec_spec, _vec_spec, _vec_spec, _mat_spec],
        out_specs=_row_spec,
        out_shape=jax.ShapeDtypeStruct((N, D), jnp.float32),
    )(s0, s1, g1, dinv, b1, lnw, lnb, w2)


def _tc_c_body(s0, s1, g2, dinv, b2, lnw, lnb, x, out):
    v = dinv[...] * (s0[...] + s1[...] + g2[...]) + b2[...]
    v = _layernorm(v, lnw[...], lnb[...]) + x[...]
    out[...] = _gelu(v)


def _tc_c(s0, s1, g2, dinv, b2, lnw, lnb, x):
    return pl.pallas_call(
        _tc_c_body,
        grid=(_GRID,),
        in_specs=[_row_spec, _row_spec, _row_spec, _col_spec,
                  _vec_spec, _vec_spec, _vec_spec, _row_spec],
        out_specs=_row_spec,
        out_shape=jax.ShapeDtypeStruct((N, D), jnp.float32),
    )(s0, s1, g2, dinv, b2, lnw, lnb, x)


def kernel(x, edge_index, edge_attr, W1, b1, ln1_w, ln1_b, W2, b2, ln2_w, ln2_b):
    row = edge_index[0]
    col = edge_index[1]
    ew = edge_attr[:, 0]

    pad = EPAD - E
    zi = jnp.zeros((pad,), jnp.int32)
    colp = jnp.concatenate([col, zi]).reshape(T, NCH, CH)
    ewp = jnp.concatenate([ew, jnp.zeros((pad,), jnp.float32)]).reshape(T, NCH, CH)

    deg2 = _deg_kernel(colp, ewp).reshape(NC, NPAD)
    deg0 = deg2[0, :N, None]
    deg1 = deg2[1, :N, None]

    g1, dinv = _tc_a(deg0, deg1, x, W1)

    epad = EEPAD - E
    ezi = jnp.zeros((epad,), jnp.int32)
    rowe = jnp.concatenate([row, ezi]).reshape(T, ENCH, 1, ECH)
    cole = jnp.concatenate([col, ezi]).reshape(T, ENCH, 1, ECH)
    rc = jnp.concatenate([rowe, cole], axis=2)
    ewe = jnp.concatenate([ew, jnp.zeros((epad,), jnp.float32)])
    ew16 = jnp.broadcast_to(ewe.reshape(T, ENCH, ECH)[..., None],
                            (T, ENCH, ECH, 16))
    g1b = jax.lax.bitcast_convert_type(
        g1[:, DPERM].astype(jnp.bfloat16).reshape(N, D // 2, 2), jnp.int32)
    s1 = _edge_kernel(rc, ew16, g1b)
    g2 = _tc_b(s1[0, :N], s1[1, :N], g1, dinv,
               b1[None, :], ln1_w[None, :], ln1_b[None, :], W2)

    g2b = jax.lax.bitcast_convert_type(
        g2[:, DPERM].astype(jnp.bfloat16).reshape(N, D // 2, 2), jnp.int32)
    s2 = _edge_kernel(rc, ew16, g2b)
    out = _tc_c(s2[0, :N], s2[1, :N], g2, dinv,
                b2[None, :], ln2_w[None, :], ln2_b[None, :], x)
    return out


# speedup vs baseline: 1.0986x; 1.0986x over previous
import functools

import jax
import jax.numpy as jnp
import numpy as np
from jax import lax
from jax.experimental import pallas as pl
from jax.experimental.pallas import tpu as pltpu
from jax.experimental.pallas import tpu_sc as plsc

N = 10000
E = 320000
D = 128

NC = 2
NS = 16
T = NC * NS

CH = 128
NCH = -(-E // (T * CH))
EPAD = T * NCH * CH

ECH = 48
ENCH = 212
NSLOT = 2
EEPAD = T * ENCH * ECH

NPAD = 10112
RPT = NPAD // NS

_l = np.arange(D)
DPERM = (_l // 32) * 32 + (_l % 32 % 2) * 16 + (_l % 32) // 2

_mesh = plsc.VectorSubcoreMesh(core_axis_name="c", subcore_axis_name="s")


@functools.partial(
    pl.kernel,
    out_type=jax.ShapeDtypeStruct((NC * NPAD,), jnp.float32),
    mesh=_mesh,
    scratch_types=[
        pltpu.VMEM((NCH, CH), jnp.int32),
        pltpu.VMEM((NCH, CH), jnp.float32),
        pltpu.VMEM((640,), jnp.float32),
        pltpu.VMEM_SHARED((NPAD,), jnp.float32),
    ],
)
def _deg_kernel(col_hbm, ew_hbm, out_hbm, col_v, ew_v, buf_v, deg_sh):
    c = lax.axis_index("c")
    s = lax.axis_index("s")
    wid = s * NC + c

    zero16 = jnp.zeros((16,), jnp.float32)
    for i in range(640 // 16):
        buf_v[pl.ds(i * 16, 16)] = zero16
    pltpu.sync_copy(buf_v.at[pl.ds(0, RPT)], deg_sh.at[pl.ds(s * RPT, RPT)])
    plsc.subcore_barrier()

    pltpu.sync_copy(col_hbm.at[wid], col_v)
    pltpu.sync_copy(ew_hbm.at[wid], ew_v)

    def chunk(ch, carry):
        pltpu.sync_copy(ew_v.at[ch], deg_sh.at[col_v.at[ch]], add=True)
        return carry

    lax.fori_loop(0, NCH, chunk, 0)
    plsc.subcore_barrier()

    pltpu.sync_copy(deg_sh.at[pl.ds(s * RPT, RPT)], buf_v.at[pl.ds(0, RPT)])
    pltpu.sync_copy(buf_v.at[pl.ds(0, RPT)],
                    out_hbm.at[pl.ds(c * NPAD + s * RPT, RPT)])


@functools.partial(
    pl.kernel,
    out_type=jax.ShapeDtypeStruct((NC, NPAD, D), jnp.float32),
    mesh=_mesh,
    scratch_types=[
        pltpu.VMEM((NSLOT, 2, ECH), jnp.int32),
        pltpu.VMEM((NSLOT, ECH, 16), jnp.float32),
        pltpu.VMEM((NSLOT, ECH, D // 2), jnp.int32),
        pltpu.VMEM((2, ECH, D), jnp.float32),
        pltpu.VMEM((2, ECH), jnp.int32),
        pltpu.VMEM_SHARED((NPAD, D), jnp.float32),
        pltpu.SemaphoreType.DMA,
        pltpu.SemaphoreType.DMA,
        pltpu.SemaphoreType.DMA,
        pltpu.SemaphoreType.DMA,
        pltpu.SemaphoreType.DMA,
        pltpu.SemaphoreType.DMA,
    ],
    compiler_params=pltpu.CompilerParams(use_tc_tiling_on_sc=False,
                                         needs_layout_passes=False),
)
def _edge_kernel(rc_hbm, ew_hbm, g_hbm, out_hbm,
                 rc_v, ew_v, rbuf, obuf, sidx, acc_sh,
                 gsem0, gsem1, psem0, psem1, ssem0, ssem1):
    c = lax.axis_index("c")
    s = lax.axis_index("s")
    wid = s * NC + c
    gsem = (gsem0, gsem1)
    psem = (psem0, psem1)
    ssem = (ssem0, ssem1)

    zero16 = jnp.zeros((16,), jnp.float32)

    def zrow(r, carry):
        for j in range(D // 16):
            obuf[0, r, pl.ds(j * 16, 16)] = zero16
        return carry

    lax.fori_loop(0, ECH, zrow, 0)
    for i in range(RPT // ECH):
        pltpu.sync_copy(obuf.at[0], acc_sh.at[pl.ds(s * RPT + i * ECH, ECH)])
    rem = RPT - (RPT // ECH) * ECH
    if rem:
        pltpu.sync_copy(obuf.at[0, pl.ds(0, rem)],
                        acc_sh.at[pl.ds(s * RPT + (RPT // ECH) * ECH, rem)])
    plsc.subcore_barrier()

    def start_pref(ch, slot):
        pltpu.async_copy(rc_hbm.at[wid, ch], rc_v.at[slot], psem[slot])
        pltpu.async_copy(ew_hbm.at[wid, ch], ew_v.at[slot], psem[slot])

    def wait_pref(slot):
        pltpu.make_async_copy(rc_hbm.at[0, 0], rc_v.at[slot], psem[slot]).wait()
        pltpu.make_async_copy(ew_hbm.at[0, 0], ew_v.at[slot], psem[slot]).wait()

    def start_gather(slot):
        pltpu.async_copy(g_hbm.at[rc_v.at[slot, 0]], rbuf.at[slot], gsem[slot])

    def wait_gather(slot):
        pltpu.make_async_copy(g_hbm.at[rc_v.at[slot, 0]], rbuf.at[slot],
                              gsem[slot]).wait()

    def wait_scatter(o):
        pltpu.make_async_copy(obuf.at[o], acc_sh.at[sidx.at[o]],
                              ssem[o]).wait()

    def scale(slot, o):
        for j in range(ECH // 16):
            sidx[o, pl.ds(j * 16, 16)] = rc_v[slot, 1, pl.ds(j * 16, 16)]

        def edge(e, ecarry):
            ews = ew_v[slot, e]
            for k in range(D // 32):
                v = rbuf[slot, e, pl.ds(k * 16, 16)]
                lo = plsc.bitcast(v << 16, jnp.float32)
                hi = plsc.bitcast(v & jnp.int32(-65536), jnp.float32)
                obuf[o, e, pl.ds(k * 32, 16)] = lo * ews
                obuf[o, e, pl.ds(k * 32 + 16, 16)] = hi * ews
            return ecarry

        lax.fori_loop(0, ECH, edge, 0)

    def start_scatter(o):
        pltpu.async_copy(obuf.at[o], acc_sh.at[sidx.at[o]], ssem[o], add=True)

    last = ENCH - 1

    for k in range(NSLOT - 1):
        start_pref(k, k)
        wait_pref(k)
        start_gather(k)
    start_pref(NSLOT - 1, NSLOT - 1)

    def ring(g, carry):
        base = NSLOT * g
        for k in range(NSLOT):
            ch = base + k
            km1 = (k + NSLOT - 1) % NSLOT
            wait_gather(k)
            wait_pref(km1)
            start_gather(km1)

            @pl.when(ch >= 2)
            def _():
                wait_scatter(k)

            scale(k, k)
            start_scatter(k)
            start_pref(jnp.minimum(ch + NSLOT, last), k)
        return carry

    lax.fori_loop(0, ENCH // NSLOT, ring, 0)
    for k in range(NSLOT - 1):
        wait_gather(k)
    wait_pref(NSLOT - 1)
    wait_scatter(0)
    wait_scatter(1)
    plsc.subcore_barrier()

    for i in range(RPT // ECH):
        pltpu.sync_copy(acc_sh.at[pl.ds(s * RPT + i * ECH, ECH)], obuf.at[0])
        pltpu.sync_copy(obuf.at[0], out_hbm.at[c, pl.ds(s * RPT + i * ECH, ECH)])
    if rem:
        pltpu.sync_copy(acc_sh.at[pl.ds(s * RPT + (RPT // ECH) * ECH, rem)],
                        obuf.at[0, pl.ds(0, rem)])
        pltpu.sync_copy(obuf.at[0, pl.ds(0, rem)],
                        out_hbm.at[c, pl.ds(s * RPT + (RPT // ECH) * ECH, rem)])


_BN = 1000
_GRID = N // _BN

_row_spec = pl.BlockSpec((_BN, D), lambda i: (i, 0))
_col_spec = pl.BlockSpec((_BN, 1), lambda i: (i, 0))
_mat_spec = pl.BlockSpec((D, D), lambda i: (0, 0))
_vec_spec = pl.BlockSpec((1, D), lambda i: (0, 0))


def _gelu(v):
    return 0.5 * v * (1.0 + lax.erf(v * 0.7071067811865476))


def _layernorm(v, w, b):
    m = jnp.mean(v, axis=-1, keepdims=True)
    var = jnp.mean((v - m) ** 2, axis=-1, keepdims=True)
    return (v - m) * lax.rsqrt(var + 1e-5) * w + b


def _tc_a_body(deg0, deg1, x, w1, g1_out, dinv_out):
    dinv = lax.rsqrt(deg0[...] + deg1[...] + 1.0)
    h = jnp.dot(x[...], w1[...], preferred_element_type=jnp.float32)
    g1_out[...] = dinv * h
    dinv_out[...] = dinv


def _tc_a(deg0, deg1, x, w1):
    return pl.pallas_call(
        _tc_a_body,
        grid=(_GRID,),
        in_specs=[_col_spec, _col_spec, _row_spec, _mat_spec],
        out_specs=[_row_spec, _col_spec],
        out_shape=[
            jax.ShapeDtypeStruct((N, D), jnp.float32),
            jax.ShapeDtypeStruct((N, 1), jnp.float32),
        ],
    )(deg0, deg1, x, w1)


def _tc_b_body(s0, s1, g1, dinv, b1, lnw, lnb, w2, g2_out):
    dv = dinv[...]
    v = dv * (s0[...] + s1[...] + g1[...]) + b1[...]
    v = _gelu(_layernorm(v, lnw[...], lnb[...]))
    g2_out[...] = dv * jnp.dot(v, w2[...], preferred_element_type=jnp.float32)


def _tc_b(s0, s1, g1, dinv, b1, lnw, lnb, w2):
    return pl.pallas_call(
        _tc_b_body,
        grid=(_GRID,),
        in_specs=[_row_spec, _row_spec, _row_spec, _col_spec,
                  _vec_spec, _vec_spec, _vec_spec, _mat_spec],
        out_specs=_row_spec,
        out_shape=jax.ShapeDtypeStruct((N, D), jnp.float32),
    )(s0, s1, g1, dinv, b1, lnw, lnb, w2)


def _tc_c_body(s0, s1, g2, dinv, b2, lnw, lnb, x, out):
    v = dinv[...] * (s0[...] + s1[...] + g2[...]) + b2[...]
    v = _layernorm(v, lnw[...], lnb[...]) + x[...]
    out[...] = _gelu(v)


def _tc_c(s0, s1, g2, dinv, b2, lnw, lnb, x):
    return pl.pallas_call(
        _tc_c_body,
        grid=(_GRID,),
        in_specs=[_row_spec, _row_spec, _row_spec, _col_spec,
                  _vec_spec, _vec_spec, _vec_spec, _row_spec],
        out_specs=_row_spec,
        out_shape=jax.ShapeDtypeStruct((N, D), jnp.float32),
    )(s0, s1, g2, dinv, b2, lnw, lnb, x)


def kernel(x, edge_index, edge_attr, W1, b1, ln1_w, ln1_b, W2, b2, ln2_w, ln2_b):
    row = edge_index[0]
    col = edge_index[1]
    ew = edge_attr[:, 0]

    pad = EPAD - E
    zi = jnp.zeros((pad,), jnp.int32)
    colp = jnp.concatenate([col, zi]).reshape(T, NCH, CH)
    ewp = jnp.concatenate([ew, jnp.zeros((pad,), jnp.float32)]).reshape(T, NCH, CH)

    deg2 = _deg_kernel(colp, ewp).reshape(NC, NPAD)
    deg0 = deg2[0, :N, None]
    deg1 = deg2[1, :N, None]

    g1, dinv = _tc_a(deg0, deg1, x, W1)

    epad = EEPAD - E
    ezi = jnp.zeros((epad,), jnp.int32)
    rowe = jnp.concatenate([row, ezi]).reshape(T, ENCH, 1, ECH)
    cole = jnp.concatenate([col, ezi]).reshape(T, ENCH, 1, ECH)
    rc = jnp.concatenate([rowe, cole], axis=2)
    ewe = jnp.concatenate([ew, jnp.zeros((epad,), jnp.float32)])
    ew16 = jnp.broadcast_to(ewe.reshape(T, ENCH, ECH)[..., None],
                            (T, ENCH, ECH, 16))
    g1b = jax.lax.bitcast_convert_type(
        g1[:, DPERM].astype(jnp.bfloat16).reshape(N, D // 2, 2), jnp.int32)
    s1 = _edge_kernel(rc, ew16, g1b)
    g2 = _tc_b(s1[0, :N], s1[1, :N], g1, dinv,
               b1[None, :], ln1_w[None, :], ln1_b[None, :], W2)

    g2b = jax.lax.bitcast_convert_type(
        g2[:, DPERM].astype(jnp.bfloat16).reshape(N, D // 2, 2), jnp.int32)
    s2 = _edge_kernel(rc, ew16, g2b)
    out = _tc_c(s2[0, :N], s2[1, :N], g2, dinv,
                b2[None, :], ln2_w[None, :], ln2_b[None, :], x)
    return out


# speedup vs baseline: 1.4428x; 1.3134x over previous
import functools

import jax
import jax.numpy as jnp
import numpy as np
from jax import lax
from jax.experimental import pallas as pl
from jax.experimental.pallas import tpu as pltpu
from jax.experimental.pallas import tpu_sc as plsc

N = 10000
E = 320000
D = 128
DH = D // 2

NC = 2
NS = 16
T = NC * NS

ECH = 48
ENCH = 212
NSLOT = 2
EPAD = T * ENCH * ECH

NPAD = 10112
RPT = NPAD // NS

_j = np.arange(DH)
PA = (_j // 16) * 32 + (_j % 16)
PB = PA + 16

_mesh = plsc.VectorSubcoreMesh(core_axis_name="c", subcore_axis_name="s")


@functools.partial(
    pl.kernel,
    out_type=jax.ShapeDtypeStruct((NC * NPAD,), jnp.float32),
    mesh=_mesh,
    scratch_types=[
        pltpu.VMEM((ENCH, ECH), jnp.int32),
        pltpu.VMEM((ENCH, ECH), jnp.float32),
        pltpu.VMEM((640,), jnp.float32),
        pltpu.VMEM_SHARED((NPAD,), jnp.float32),
    ],
)
def _deg_kernel(col_hbm, ew_hbm, out_hbm, col_v, ew_v, buf_v, deg_sh):
    c = lax.axis_index("c")
    s = lax.axis_index("s")
    wid = s * NC + c

    zero16 = jnp.zeros((16,), jnp.float32)
    for i in range(640 // 16):
        buf_v[pl.ds(i * 16, 16)] = zero16
    pltpu.sync_copy(buf_v.at[pl.ds(0, RPT)], deg_sh.at[pl.ds(s * RPT, RPT)])
    plsc.subcore_barrier()

    pltpu.sync_copy(col_hbm.at[wid], col_v)
    pltpu.sync_copy(ew_hbm.at[wid], ew_v)

    def chunk(ch, carry):
        pltpu.sync_copy(ew_v.at[ch], deg_sh.at[col_v.at[ch]], add=True)
        return carry

    lax.fori_loop(0, ENCH, chunk, 0)
    plsc.subcore_barrier()

    pltpu.sync_copy(deg_sh.at[pl.ds(s * RPT, RPT)], buf_v.at[pl.ds(0, RPT)])
    pltpu.sync_copy(buf_v.at[pl.ds(0, RPT)],
                    out_hbm.at[pl.ds(c * NPAD + s * RPT, RPT)])


@functools.partial(
    pl.kernel,
    out_type=jax.ShapeDtypeStruct((NC, NPAD, D), jnp.float32),
    mesh=_mesh,
    scratch_types=[
        pltpu.VMEM((NSLOT, ECH), jnp.int32),
        pltpu.VMEM((NSLOT, ECH), jnp.int32),
        pltpu.VMEM((NSLOT, ECH), jnp.float32),
        pltpu.VMEM((NSLOT, ECH, DH), jnp.int32),
        pltpu.VMEM((NSLOT, ECH, D), jnp.float32),
        pltpu.VMEM((NSLOT, ECH), jnp.int32),
        pltpu.VMEM_SHARED((NPAD, D), jnp.float32),
        pltpu.SemaphoreType.DMA,
        pltpu.SemaphoreType.DMA,
        pltpu.SemaphoreType.DMA,
        pltpu.SemaphoreType.DMA,
        pltpu.SemaphoreType.DMA,
        pltpu.SemaphoreType.DMA,
    ],
    compiler_params=pltpu.CompilerParams(use_tc_tiling_on_sc=False,
                                         needs_layout_passes=False),
)
def _edge_kernel(row_hbm, col_hbm, ew_hbm, g_hbm, out_hbm,
                 row_v, col_v, ew_v, rbuf, obuf, sidx, acc_sh,
                 gsem0, gsem1, psem0, psem1, ssem0, ssem1):
    c = lax.axis_index("c")
    s = lax.axis_index("s")
    wid = s * NC + c
    gsem = (gsem0, gsem1)
    psem = (psem0, psem1)
    ssem = (ssem0, ssem1)

    zero16 = jnp.zeros((16,), jnp.float32)

    def zrow(r, carry):
        for j in range(D // 16):
            obuf[0, r, pl.ds(j * 16, 16)] = zero16
        return carry

    lax.fori_loop(0, ECH, zrow, 0)
    for i in range(RPT // ECH):
        pltpu.sync_copy(obuf.at[0], acc_sh.at[pl.ds(s * RPT + i * ECH, ECH)])
    rem = RPT - (RPT // ECH) * ECH
    if rem:
        pltpu.sync_copy(obuf.at[0, pl.ds(0, rem)],
                        acc_sh.at[pl.ds(s * RPT + (RPT // ECH) * ECH, rem)])
    plsc.subcore_barrier()

    def start_pref(ch, slot):
        pltpu.async_copy(row_hbm.at[wid, ch], row_v.at[slot], psem[slot])
        pltpu.async_copy(col_hbm.at[wid, ch], col_v.at[slot], psem[slot])
        pltpu.async_copy(ew_hbm.at[wid, ch], ew_v.at[slot], psem[slot])

    def wait_pref(slot):
        pltpu.make_async_copy(row_hbm.at[0, 0], row_v.at[slot], psem[slot]).wait()
        pltpu.make_async_copy(col_hbm.at[0, 0], col_v.at[slot], psem[slot]).wait()
        pltpu.make_async_copy(ew_hbm.at[0, 0], ew_v.at[slot], psem[slot]).wait()

    def start_gather(slot):
        pltpu.async_copy(g_hbm.at[row_v.at[slot]], rbuf.at[slot], gsem[slot])

    def wait_gather(slot):
        pltpu.make_async_copy(g_hbm.at[row_v.at[slot]], rbuf.at[slot],
                              gsem[slot]).wait()

    def wait_scatter(o):
        pltpu.make_async_copy(obuf.at[o], acc_sh.at[sidx.at[o]],
                              ssem[o]).wait()

    def scale(k):
        for j in range(ECH // 16):
            sidx[k, pl.ds(j * 16, 16)] = col_v[k, pl.ds(j * 16, 16)]
        kvec = jnp.full((16,), k, jnp.int32)

        def edge(e, ecarry):
            ews = plsc.load_gather(ew_v, [kvec, jnp.full((16,), e, jnp.int32)])
            for q in range(D // 32):
                v = rbuf[k, e, pl.ds(q * 16, 16)]
                lo = plsc.bitcast(v << 16, jnp.float32)
                hi = plsc.bitcast(v & jnp.int32(-65536), jnp.float32)
                obuf[k, e, pl.ds(q * 32, 16)] = lo * ews
                obuf[k, e, pl.ds(q * 32 + 16, 16)] = hi * ews
            return ecarry

        lax.fori_loop(0, ECH, edge, 0)

    def start_scatter(o):
        pltpu.async_copy(obuf.at[o], acc_sh.at[sidx.at[o]], ssem[o], add=True)

    last = ENCH - 1

    start_pref(0, 0)
    wait_pref(0)
    start_gather(0)
    start_pref(1, 1)

    def ring(g, carry):
        base = NSLOT * g
        for k in range(NSLOT):
            ch = base + k
            km1 = (k + NSLOT - 1) % NSLOT
            wait_gather(k)
            wait_pref(km1)
            start_gather(km1)

            @pl.when(ch >= 2)
            def _():
                wait_scatter(k)

            scale(k)
            start_scatter(k)
            start_pref(jnp.minimum(ch + NSLOT, last), k)
        return carry

    lax.fori_loop(0, ENCH // NSLOT, ring, 0)
    wait_gather(0)
    wait_pref(1)
    wait_scatter(0)
    wait_scatter(1)
    plsc.subcore_barrier()

    for i in range(RPT // ECH):
        pltpu.sync_copy(acc_sh.at[pl.ds(s * RPT + i * ECH, ECH)], obuf.at[0])
        pltpu.sync_copy(obuf.at[0], out_hbm.at[c, pl.ds(s * RPT + i * ECH, ECH)])
    if rem:
        pltpu.sync_copy(acc_sh.at[pl.ds(s * RPT + (RPT // ECH) * ECH, rem)],
                        obuf.at[0, pl.ds(0, rem)])
        pltpu.sync_copy(obuf.at[0, pl.ds(0, rem)],
                        out_hbm.at[c, pl.ds(s * RPT + (RPT // ECH) * ECH, rem)])


_BN = 1000
_GRID = N // _BN

_row_spec = pl.BlockSpec((_BN, D), lambda i: (i, 0))
_half_spec = pl.BlockSpec((_BN, DH), lambda i: (i, 0))
_col_spec = pl.BlockSpec((_BN, 1), lambda i: (i, 0))
_mat_spec = pl.BlockSpec((D, D), lambda i: (0, 0))
_hmat_spec = pl.BlockSpec((D, DH), lambda i: (0, 0))
_vec_spec = pl.BlockSpec((1, D), lambda i: (0, 0))
_s0_spec = pl.BlockSpec((1, _BN, D), lambda i: (0, i, 0))
_s1_spec = pl.BlockSpec((1, _BN, D), lambda i: (1, i, 0))
_d0_spec = pl.BlockSpec((1, _BN, 1), lambda i: (0, i, 0))
_d1_spec = pl.BlockSpec((1, _BN, 1), lambda i: (1, i, 0))


def _gelu(v):
    return 0.5 * v * (1.0 + lax.erf(v * 0.7071067811865476))


def _layernorm(v, w, b):
    m = jnp.mean(v, axis=-1, keepdims=True)
    var = jnp.mean((v - m) ** 2, axis=-1, keepdims=True)
    return (v - m) * lax.rsqrt(var + 1e-5) * w + b


def _pack_bf16_pair(a, b):
    ba = lax.bitcast_convert_type(a, jnp.int32)
    bb = lax.bitcast_convert_type(b, jnp.int32)
    ra = ba + 32767 + (lax.shift_right_logical(ba, 16) & 1)
    rb = bb + 32767 + (lax.shift_right_logical(bb, 16) & 1)
    return lax.shift_right_logical(ra, 16) | (rb & jnp.int32(-65536))


def _tc_a_body(deg0, deg1, x, w1, w1a, w1b, g1_out, g1p_out, dinv_out):
    dinv = lax.rsqrt(deg0[0] + deg1[0] + 1.0)
    xv = x[...]
    g1_out[...] = dinv * jnp.dot(xv, w1[...], preferred_element_type=jnp.float32)
    pa = dinv * jnp.dot(xv, w1a[...], preferred_element_type=jnp.float32)
    pb = dinv * jnp.dot(xv, w1b[...], preferred_element_type=jnp.float32)
    g1p_out[...] = _pack_bf16_pair(pa, pb)
    dinv_out[...] = dinv


def _tc_a(deg3, x, w1, w1a, w1b):
    return pl.pallas_call(
        _tc_a_body,
        grid=(_GRID,),
        in_specs=[_d0_spec, _d1_spec, _row_spec, _mat_spec,
                  _hmat_spec, _hmat_spec],
        out_specs=[_row_spec, _half_spec, _col_spec],
        out_shape=[
            jax.ShapeDtypeStruct((N, D), jnp.float32),
            jax.ShapeDtypeStruct((N, DH), jnp.int32),
            jax.ShapeDtypeStruct((N, 1), jnp.float32),
        ],
    )(deg3, deg3, x, w1, w1a, w1b)


def _tc_b_body(s1, s1b, g1, dinv, b1, lnw, lnb, w2, w2a, w2b,
               g2_out, g2p_out):
    dv = dinv[...]
    v = dv * (s1[0] + s1b[0] + g1[...]) + b1[...]
    v = _gelu(_layernorm(v, lnw[...], lnb[...]))
    g2_out[...] = dv * jnp.dot(v, w2[...], preferred_element_type=jnp.float32)
    pa = dv * jnp.dot(v, w2a[...], preferred_element_type=jnp.float32)
    pb = dv * jnp.dot(v, w2b[...], preferred_element_type=jnp.float32)
    g2p_out[...] = _pack_bf16_pair(pa, pb)


def _tc_b(s1, g1, dinv, b1, lnw, lnb, w2, w2a, w2b):
    return pl.pallas_call(
        _tc_b_body,
        grid=(_GRID,),
        in_specs=[_s0_spec, _s1_spec, _row_spec, _col_spec,
                  _vec_spec, _vec_spec, _vec_spec, _mat_spec,
                  _hmat_spec, _hmat_spec],
        out_specs=[_row_spec, _half_spec],
        out_shape=[
            jax.ShapeDtypeStruct((N, D), jnp.float32),
            jax.ShapeDtypeStruct((N, DH), jnp.int32),
        ],
    )(s1, s1, g1, dinv, b1, lnw, lnb, w2, w2a, w2b)


def _tc_c_body(s2, s2b, g2, dinv, b2, lnw, lnb, x, out):
    v = dinv[...] * (s2[0] + s2b[0] + g2[...]) + b2[...]
    v = _layernorm(v, lnw[...], lnb[...]) + x[...]
    out[...] = _gelu(v)


def _tc_c(s2, g2, dinv, b2, lnw, lnb, x):
    return pl.pallas_call(
        _tc_c_body,
        grid=(_GRID,),
        in_specs=[_s0_spec, _s1_spec, _row_spec, _col_spec,
                  _vec_spec, _vec_spec, _vec_spec, _row_spec],
        out_specs=_row_spec,
        out_shape=jax.ShapeDtypeStruct((N, D), jnp.float32),
    )(s2, s2, g2, dinv, b2, lnw, lnb, x)


def kernel(x, edge_index, edge_attr, W1, b1, ln1_w, ln1_b, W2, b2, ln2_w, ln2_b):
    row = edge_index[0]
    col = edge_index[1]
    ew = edge_attr[:, 0]

    pad = EPAD - E
    zi = jnp.zeros((pad,), jnp.int32)
    rowp = jnp.concatenate([row, zi]).reshape(T, ENCH, ECH)
    colp = jnp.concatenate([col, zi]).reshape(T, ENCH, ECH)
    ewp = jnp.concatenate([ew, jnp.zeros((pad,), jnp.float32)]).reshape(
        T, ENCH, ECH)

    deg3 = _deg_kernel(colp, ewp).reshape(NC, NPAD, 1)

    g1, g1p, dinv = _tc_a(deg3, x, W1, W1[:, PA], W1[:, PB])

    s1 = _edge_kernel(rowp, colp, ewp, g1p)
    g2, g2p = _tc_b(s1, g1, dinv,
                    b1[None, :], ln1_w[None, :], ln1_b[None, :],
                    W2, W2[:, PA], W2[:, PB])

    s2 = _edge_kernel(rowp, colp, ewp, g2p)
    out = _tc_c(s2, g2, dinv,
                b2[None, :], ln2_w[None, :], ln2_b[None, :], x)
    return out


# speedup vs baseline: 1.5793x; 1.0946x over previous
import functools

import jax
import jax.numpy as jnp
import numpy as np
from jax import lax
from jax.experimental import pallas as pl
from jax.experimental.pallas import tpu as pltpu
from jax.experimental.pallas import tpu_sc as plsc

N = 10000
E = 320000
D = 128
DH = D // 2

NC = 2
NS = 16
T = NC * NS

ECH = 64
ENCH = 158
NSLOT = 2
EPAD = T * ENCH * ECH

NPAD = 10112
RPT = NPAD // NS

_j = np.arange(DH)
PA = (_j // 16) * 32 + (_j % 16)
PB = PA + 16

_mesh = plsc.VectorSubcoreMesh(core_axis_name="c", subcore_axis_name="s")


@functools.partial(
    pl.kernel,
    out_type=jax.ShapeDtypeStruct((NC * NPAD,), jnp.float32),
    mesh=_mesh,
    scratch_types=[
        pltpu.VMEM((ENCH, ECH), jnp.int32),
        pltpu.VMEM((ENCH, ECH), jnp.float32),
        pltpu.VMEM((640,), jnp.float32),
        pltpu.VMEM_SHARED((NPAD,), jnp.float32),
    ],
)
def _deg_kernel(col_hbm, ew_hbm, out_hbm, col_v, ew_v, buf_v, deg_sh):
    c = lax.axis_index("c")
    s = lax.axis_index("s")
    wid = s * NC + c

    zero16 = jnp.zeros((16,), jnp.float32)
    for i in range(640 // 16):
        buf_v[pl.ds(i * 16, 16)] = zero16
    pltpu.sync_copy(buf_v.at[pl.ds(0, RPT)], deg_sh.at[pl.ds(s * RPT, RPT)])
    plsc.subcore_barrier()

    pltpu.sync_copy(col_hbm.at[wid], col_v)
    pltpu.sync_copy(ew_hbm.at[wid], ew_v)

    def chunk(ch, carry):
        pltpu.sync_copy(ew_v.at[ch], deg_sh.at[col_v.at[ch]], add=True)
        return carry

    lax.fori_loop(0, ENCH, chunk, 0)
    plsc.subcore_barrier()

    pltpu.sync_copy(deg_sh.at[pl.ds(s * RPT, RPT)], buf_v.at[pl.ds(0, RPT)])
    pltpu.sync_copy(buf_v.at[pl.ds(0, RPT)],
                    out_hbm.at[pl.ds(c * NPAD + s * RPT, RPT)])


@functools.partial(
    pl.kernel,
    out_type=jax.ShapeDtypeStruct((NC, NPAD, D), jnp.float32),
    mesh=_mesh,
    scratch_types=[
        pltpu.VMEM((NSLOT, ECH), jnp.int32),
        pltpu.VMEM((NSLOT, ECH), jnp.int32),
        pltpu.VMEM((NSLOT, ECH), jnp.float32),
        pltpu.VMEM((NSLOT, ECH, DH), jnp.int32),
        pltpu.VMEM((NSLOT, ECH, D), jnp.float32),
        pltpu.VMEM((NSLOT, ECH), jnp.int32),
        pltpu.VMEM_SHARED((NPAD, D), jnp.float32),
        pltpu.SemaphoreType.DMA,
        pltpu.SemaphoreType.DMA,
        pltpu.SemaphoreType.DMA,
        pltpu.SemaphoreType.DMA,
        pltpu.SemaphoreType.DMA,
        pltpu.SemaphoreType.DMA,
    ],
    compiler_params=pltpu.CompilerParams(use_tc_tiling_on_sc=False,
                                         needs_layout_passes=False),
)
def _edge_kernel(row_hbm, col_hbm, ew_hbm, g_hbm, out_hbm,
                 row_v, col_v, ew_v, rbuf, obuf, sidx, acc_sh,
                 gsem0, gsem1, psem0, psem1, ssem0, ssem1):
    c = lax.axis_index("c")
    s = lax.axis_index("s")
    wid = s * NC + c
    gsem = (gsem0, gsem1)
    psem = (psem0, psem1)
    ssem = (ssem0, ssem1)

    zero16 = jnp.zeros((16,), jnp.float32)

    def zrow(r, carry):
        for j in range(D // 16):
            obuf[0, r, pl.ds(j * 16, 16)] = zero16
        return carry

    lax.fori_loop(0, ECH, zrow, 0)
    for i in range(RPT // ECH):
        pltpu.sync_copy(obuf.at[0], acc_sh.at[pl.ds(s * RPT + i * ECH, ECH)])
    rem = RPT - (RPT // ECH) * ECH
    if rem:
        pltpu.sync_copy(obuf.at[0, pl.ds(0, rem)],
                        acc_sh.at[pl.ds(s * RPT + (RPT // ECH) * ECH, rem)])
    plsc.subcore_barrier()

    def start_pref(ch, slot):
        pltpu.async_copy(row_hbm.at[wid, ch], row_v.at[slot], psem[slot])
        pltpu.async_copy(col_hbm.at[wid, ch], col_v.at[slot], psem[slot])
        pltpu.async_copy(ew_hbm.at[wid, ch], ew_v.at[slot], psem[slot])

    def wait_pref(slot):
        pltpu.make_async_copy(row_hbm.at[0, 0], row_v.at[slot], psem[slot]).wait()
        pltpu.make_async_copy(col_hbm.at[0, 0], col_v.at[slot], psem[slot]).wait()
        pltpu.make_async_copy(ew_hbm.at[0, 0], ew_v.at[slot], psem[slot]).wait()

    def start_gather(slot):
        pltpu.async_copy(g_hbm.at[row_v.at[slot]], rbuf.at[slot], gsem[slot])

    def wait_gather(slot):
        pltpu.make_async_copy(g_hbm.at[row_v.at[slot]], rbuf.at[slot],
                              gsem[slot]).wait()

    def wait_scatter(o):
        pltpu.make_async_copy(obuf.at[o], acc_sh.at[sidx.at[o]],
                              ssem[o]).wait()

    def scale(k):
        for j in range(ECH // 16):
            sidx[k, pl.ds(j * 16, 16)] = col_v[k, pl.ds(j * 16, 16)]
        kvec = jnp.full((16,), k, jnp.int32)

        def edge(e, ecarry):
            ews = plsc.load_gather(ew_v, [kvec, jnp.full((16,), e, jnp.int32)])
            for q in range(D // 32):
                v = rbuf[k, e, pl.ds(q * 16, 16)]
                lo = plsc.bitcast(v << 16, jnp.float32)
                hi = plsc.bitcast(v & jnp.int32(-65536), jnp.float32)
                obuf[k, e, pl.ds(q * 32, 16)] = lo * ews
                obuf[k, e, pl.ds(q * 32 + 16, 16)] = hi * ews
            return ecarry

        lax.fori_loop(0, ECH, edge, 0)

    def start_scatter(o):
        pltpu.async_copy(obuf.at[o], acc_sh.at[sidx.at[o]], ssem[o], add=True)

    last = ENCH - 1

    start_pref(0, 0)
    wait_pref(0)
    start_gather(0)
    start_pref(1, 1)

    def ring(g, carry):
        base = NSLOT * g
        for k in range(NSLOT):
            ch = base + k
            km1 = (k + NSLOT - 1) % NSLOT
            wait_gather(k)
            wait_pref(km1)
            start_gather(km1)

            @pl.when(ch >= 2)
            def _():
                wait_scatter(k)

            scale(k)
            start_scatter(k)
            start_pref(jnp.minimum(ch + NSLOT, last), k)
        return carry

    lax.fori_loop(0, ENCH // NSLOT, ring, 0)
    wait_gather(0)
    wait_pref(1)
    wait_scatter(0)
    wait_scatter(1)
    plsc.subcore_barrier()

    for i in range(RPT // ECH):
        pltpu.sync_copy(acc_sh.at[pl.ds(s * RPT + i * ECH, ECH)], obuf.at[0])
        pltpu.sync_copy(obuf.at[0], out_hbm.at[c, pl.ds(s * RPT + i * ECH, ECH)])
    if rem:
        pltpu.sync_copy(acc_sh.at[pl.ds(s * RPT + (RPT // ECH) * ECH, rem)],
                        obuf.at[0, pl.ds(0, rem)])
        pltpu.sync_copy(obuf.at[0, pl.ds(0, rem)],
                        out_hbm.at[c, pl.ds(s * RPT + (RPT // ECH) * ECH, rem)])


_BN = 1000
_GRID = N // _BN

_row_spec = pl.BlockSpec((_BN, D), lambda i: (i, 0))
_half_spec = pl.BlockSpec((_BN, DH), lambda i: (i, 0))
_col_spec = pl.BlockSpec((_BN, 1), lambda i: (i, 0))
_mat_spec = pl.BlockSpec((D, D), lambda i: (0, 0))
_hmat_spec = pl.BlockSpec((D, DH), lambda i: (0, 0))
_vec_spec = pl.BlockSpec((1, D), lambda i: (0, 0))
_s0_spec = pl.BlockSpec((1, _BN, D), lambda i: (0, i, 0))
_s1_spec = pl.BlockSpec((1, _BN, D), lambda i: (1, i, 0))
_d0_spec = pl.BlockSpec((1, _BN, 1), lambda i: (0, i, 0))
_d1_spec = pl.BlockSpec((1, _BN, 1), lambda i: (1, i, 0))


def _gelu(v):
    return 0.5 * v * (1.0 + lax.erf(v * 0.7071067811865476))


def _layernorm(v, w, b):
    m = jnp.mean(v, axis=-1, keepdims=True)
    var = jnp.mean((v - m) ** 2, axis=-1, keepdims=True)
    return (v - m) * lax.rsqrt(var + 1e-5) * w + b


def _pack_bf16_pair(a, b):
    ba = lax.bitcast_convert_type(a, jnp.int32)
    bb = lax.bitcast_convert_type(b, jnp.int32)
    ra = ba + 32767 + (lax.shift_right_logical(ba, 16) & 1)
    rb = bb + 32767 + (lax.shift_right_logical(bb, 16) & 1)
    return lax.shift_right_logical(ra, 16) | (rb & jnp.int32(-65536))


def _tc_a_body(deg0, deg1, x, w1, w1a, w1b, g1_out, g1p_out, dinv_out):
    dinv = lax.rsqrt(deg0[0] + deg1[0] + 1.0)
    xv = x[...]
    g1_out[...] = dinv * jnp.dot(xv, w1[...], preferred_element_type=jnp.float32)
    pa = dinv * jnp.dot(xv, w1a[...], preferred_element_type=jnp.float32)
    pb = dinv * jnp.dot(xv, w1b[...], preferred_element_type=jnp.float32)
    g1p_out[...] = _pack_bf16_pair(pa, pb)
    dinv_out[...] = dinv


def _tc_a(deg3, x, w1, w1a, w1b):
    return pl.pallas_call(
        _tc_a_body,
        grid=(_GRID,),
        in_specs=[_d0_spec, _d1_spec, _row_spec, _mat_spec,
                  _hmat_spec, _hmat_spec],
        out_specs=[_row_spec, _half_spec, _col_spec],
        out_shape=[
            jax.ShapeDtypeStruct((N, D), jnp.float32),
            jax.ShapeDtypeStruct((N, DH), jnp.int32),
            jax.ShapeDtypeStruct((N, 1), jnp.float32),
        ],
    )(deg3, deg3, x, w1, w1a, w1b)


def _tc_b_body(s1, s1b, g1, dinv, b1, lnw, lnb, w2, w2a, w2b,
               g2_out, g2p_out):
    dv = dinv[...]
    v = dv * (s1[0] + s1b[0] + g1[...]) + b1[...]
    v = _gelu(_layernorm(v, lnw[...], lnb[...]))
    g2_out[...] = dv * jnp.dot(v, w2[...], preferred_element_type=jnp.float32)
    pa = dv * jnp.dot(v, w2a[...], preferred_element_type=jnp.float32)
    pb = dv * jnp.dot(v, w2b[...], preferred_element_type=jnp.float32)
    g2p_out[...] = _pack_bf16_pair(pa, pb)


def _tc_b(s1, g1, dinv, b1, lnw, lnb, w2, w2a, w2b):
    return pl.pallas_call(
        _tc_b_body,
        grid=(_GRID,),
        in_specs=[_s0_spec, _s1_spec, _row_spec, _col_spec,
                  _vec_spec, _vec_spec, _vec_spec, _mat_spec,
                  _hmat_spec, _hmat_spec],
        out_specs=[_row_spec, _half_spec],
        out_shape=[
            jax.ShapeDtypeStruct((N, D), jnp.float32),
            jax.ShapeDtypeStruct((N, DH), jnp.int32),
        ],
    )(s1, s1, g1, dinv, b1, lnw, lnb, w2, w2a, w2b)


def _tc_c_body(s2, s2b, g2, dinv, b2, lnw, lnb, x, out):
    v = dinv[...] * (s2[0] + s2b[0] + g2[...]) + b2[...]
    v = _layernorm(v, lnw[...], lnb[...]) + x[...]
    out[...] = _gelu(v)


def _tc_c(s2, g2, dinv, b2, lnw, lnb, x):
    return pl.pallas_call(
        _tc_c_body,
        grid=(_GRID,),
        in_specs=[_s0_spec, _s1_spec, _row_spec, _col_spec,
                  _vec_spec, _vec_spec, _vec_spec, _row_spec],
        out_specs=_row_spec,
        out_shape=jax.ShapeDtypeStruct((N, D), jnp.float32),
    )(s2, s2, g2, dinv, b2, lnw, lnb, x)


def kernel(x, edge_index, edge_attr, W1, b1, ln1_w, ln1_b, W2, b2, ln2_w, ln2_b):
    row = edge_index[0]
    col = edge_index[1]
    ew = edge_attr[:, 0]

    pad = EPAD - E
    zi = jnp.zeros((pad,), jnp.int32)
    rowp = jnp.concatenate([row, zi]).reshape(T, ENCH, ECH)
    colp = jnp.concatenate([col, zi]).reshape(T, ENCH, ECH)
    ewp = jnp.concatenate([ew, jnp.zeros((pad,), jnp.float32)]).reshape(
        T, ENCH, ECH)

    deg3 = _deg_kernel(colp, ewp).reshape(NC, NPAD, 1)

    g1, g1p, dinv = _tc_a(deg3, x, W1, W1[:, PA], W1[:, PB])

    s1 = _edge_kernel(rowp, colp, ewp, g1p)
    g2, g2p = _tc_b(s1, g1, dinv,
                    b1[None, :], ln1_w[None, :], ln1_b[None, :],
                    W2, W2[:, PA], W2[:, PB])

    s2 = _edge_kernel(rowp, colp, ewp, g2p)
    out = _tc_c(s2, g2, dinv,
                b2[None, :], ln2_w[None, :], ln2_b[None, :], x)
    return out


# speedup vs baseline: 1.6279x; 1.0308x over previous
import functools

import jax
import jax.numpy as jnp
import numpy as np
from jax import lax
from jax.experimental import pallas as pl
from jax.experimental.pallas import tpu as pltpu
from jax.experimental.pallas import tpu_sc as plsc

N = 10000
E = 320000
D = 128
DH = D // 2

NC = 2
NS = 16
T = NC * NS

ECH = 96
SUB = 48
ENCH = 106
NSLOT = 2
EPAD = T * ENCH * ECH

NPAD = 10112
RPT = NPAD // NS

_j = np.arange(DH)
PA = (_j // 16) * 32 + (_j % 16)
PB = PA + 16

_mesh = plsc.VectorSubcoreMesh(core_axis_name="c", subcore_axis_name="s")


@functools.partial(
    pl.kernel,
    out_type=jax.ShapeDtypeStruct((NC * NPAD,), jnp.float32),
    mesh=_mesh,
    scratch_types=[
        pltpu.VMEM((ENCH, ECH), jnp.int32),
        pltpu.VMEM((ENCH, ECH), jnp.float32),
        pltpu.VMEM((640,), jnp.float32),
        pltpu.VMEM_SHARED((NPAD,), jnp.float32),
    ],
)
def _deg_kernel(col_hbm, ew_hbm, out_hbm, col_v, ew_v, buf_v, deg_sh):
    c = lax.axis_index("c")
    s = lax.axis_index("s")
    wid = s * NC + c

    zero16 = jnp.zeros((16,), jnp.float32)
    for i in range(640 // 16):
        buf_v[pl.ds(i * 16, 16)] = zero16
    pltpu.sync_copy(buf_v.at[pl.ds(0, RPT)], deg_sh.at[pl.ds(s * RPT, RPT)])
    plsc.subcore_barrier()

    pltpu.sync_copy(col_hbm.at[wid], col_v)
    pltpu.sync_copy(ew_hbm.at[wid], ew_v)

    def chunk(ch, carry):
        pltpu.sync_copy(ew_v.at[ch], deg_sh.at[col_v.at[ch]], add=True)
        return carry

    lax.fori_loop(0, ENCH, chunk, 0)
    plsc.subcore_barrier()

    pltpu.sync_copy(deg_sh.at[pl.ds(s * RPT, RPT)], buf_v.at[pl.ds(0, RPT)])
    pltpu.sync_copy(buf_v.at[pl.ds(0, RPT)],
                    out_hbm.at[pl.ds(c * NPAD + s * RPT, RPT)])


@functools.partial(
    pl.kernel,
    out_type=jax.ShapeDtypeStruct((NC, NPAD, D), jnp.float32),
    mesh=_mesh,
    scratch_types=[
        pltpu.VMEM((NSLOT, ECH), jnp.int32),
        pltpu.VMEM((NSLOT, ECH), jnp.int32),
        pltpu.VMEM((NSLOT, ECH), jnp.float32),
        pltpu.VMEM((NSLOT, ECH, DH), jnp.int32),
        pltpu.VMEM((2, SUB, D), jnp.float32),
        pltpu.VMEM((2, SUB), jnp.int32),
        pltpu.VMEM_SHARED((NPAD, D), jnp.float32),
        pltpu.SemaphoreType.DMA,
        pltpu.SemaphoreType.DMA,
        pltpu.SemaphoreType.DMA,
        pltpu.SemaphoreType.DMA,
        pltpu.SemaphoreType.DMA,
        pltpu.SemaphoreType.DMA,
    ],
    compiler_params=pltpu.CompilerParams(use_tc_tiling_on_sc=False,
                                         needs_layout_passes=False),
)
def _edge_kernel(row_hbm, col_hbm, ew_hbm, g_hbm, out_hbm,
                 row_v, col_v, ew_v, rbuf, obuf, sidx, acc_sh,
                 gsem0, gsem1, psem0, psem1, ssem0, ssem1):
    c = lax.axis_index("c")
    s = lax.axis_index("s")
    wid = s * NC + c
    gsem = (gsem0, gsem1)
    psem = (psem0, psem1)
    ssem = (ssem0, ssem1)

    zero16 = jnp.zeros((16,), jnp.float32)

    def zrow(r, carry):
        for j in range(D // 16):
            obuf[0, r, pl.ds(j * 16, 16)] = zero16
        return carry

    lax.fori_loop(0, SUB, zrow, 0)
    for i in range(RPT // SUB):
        pltpu.sync_copy(obuf.at[0], acc_sh.at[pl.ds(s * RPT + i * SUB, SUB)])
    rem = RPT - (RPT // SUB) * SUB
    if rem:
        pltpu.sync_copy(obuf.at[0, pl.ds(0, rem)],
                        acc_sh.at[pl.ds(s * RPT + (RPT // SUB) * SUB, rem)])
    plsc.subcore_barrier()

    def start_pref(ch, slot):
        pltpu.async_copy(row_hbm.at[wid, ch], row_v.at[slot], psem[slot])
        pltpu.async_copy(col_hbm.at[wid, ch], col_v.at[slot], psem[slot])
        pltpu.async_copy(ew_hbm.at[wid, ch], ew_v.at[slot], psem[slot])

    def wait_pref(slot):
        pltpu.make_async_copy(row_hbm.at[0, 0], row_v.at[slot], psem[slot]).wait()
        pltpu.make_async_copy(col_hbm.at[0, 0], col_v.at[slot], psem[slot]).wait()
        pltpu.make_async_copy(ew_hbm.at[0, 0], ew_v.at[slot], psem[slot]).wait()

    def start_gather(slot):
        pltpu.async_copy(g_hbm.at[row_v.at[slot]], rbuf.at[slot], gsem[slot])

    def wait_gather(slot):
        pltpu.make_async_copy(g_hbm.at[row_v.at[slot]], rbuf.at[slot],
                              gsem[slot]).wait()

    def wait_scatter(o):
        pltpu.make_async_copy(obuf.at[o], acc_sh.at[sidx.at[o]],
                              ssem[o]).wait()

    def scale_half(k, h):
        for j in range(SUB // 16):
            sidx[h, pl.ds(j * 16, 16)] = col_v[k, pl.ds(h * SUB + j * 16, 16)]
        kvec = jnp.full((16,), k, jnp.int32)

        def edge(e, ecarry):
            ews = plsc.load_gather(
                ew_v, [kvec, jnp.full((16,), h * SUB, jnp.int32) + e])
            for q in range(D // 32):
                v = rbuf[k, h * SUB + e, pl.ds(q * 16, 16)]
                lo = plsc.bitcast(v << 16, jnp.float32)
                hi = plsc.bitcast(v & jnp.int32(-65536), jnp.float32)
                obuf[h, e, pl.ds(q * 32, 16)] = lo * ews
                obuf[h, e, pl.ds(q * 32 + 16, 16)] = hi * ews
            return ecarry

        lax.fori_loop(0, SUB, edge, 0)

    def start_scatter(o):
        pltpu.async_copy(obuf.at[o], acc_sh.at[sidx.at[o]], ssem[o], add=True)

    last = ENCH - 1

    start_pref(0, 0)
    wait_pref(0)
    start_gather(0)
    start_pref(1, 1)

    def ring(g, carry):
        base = NSLOT * g
        for k in range(NSLOT):
            ch = base + k
            km1 = (k + NSLOT - 1) % NSLOT
            wait_gather(k)
            wait_pref(km1)
            start_gather(km1)
            for h in range(2):
                @pl.when(ch >= 1)
                def _():
                    wait_scatter(h)

                scale_half(k, h)
                start_scatter(h)
            start_pref(jnp.minimum(ch + NSLOT, last), k)
        return carry

    lax.fori_loop(0, ENCH // NSLOT, ring, 0)
    wait_gather(0)
    wait_pref(1)
    wait_scatter(0)
    wait_scatter(1)
    plsc.subcore_barrier()

    for i in range(RPT // SUB):
        pltpu.sync_copy(acc_sh.at[pl.ds(s * RPT + i * SUB, SUB)], obuf.at[0])
        pltpu.sync_copy(obuf.at[0], out_hbm.at[c, pl.ds(s * RPT + i * SUB, SUB)])
    if rem:
        pltpu.sync_copy(acc_sh.at[pl.ds(s * RPT + (RPT // SUB) * SUB, rem)],
                        obuf.at[0, pl.ds(0, rem)])
        pltpu.sync_copy(obuf.at[0, pl.ds(0, rem)],
                        out_hbm.at[c, pl.ds(s * RPT + (RPT // SUB) * SUB, rem)])


_BN = 1000
_GRID = N // _BN

_row_spec = pl.BlockSpec((_BN, D), lambda i: (i, 0))
_half_spec = pl.BlockSpec((_BN, DH), lambda i: (i, 0))
_col_spec = pl.BlockSpec((_BN, 1), lambda i: (i, 0))
_mat_spec = pl.BlockSpec((D, D), lambda i: (0, 0))
_hmat_spec = pl.BlockSpec((D, DH), lambda i: (0, 0))
_vec_spec = pl.BlockSpec((1, D), lambda i: (0, 0))
_s0_spec = pl.BlockSpec((1, _BN, D), lambda i: (0, i, 0))
_s1_spec = pl.BlockSpec((1, _BN, D), lambda i: (1, i, 0))
_d0_spec = pl.BlockSpec((1, _BN, 1), lambda i: (0, i, 0))
_d1_spec = pl.BlockSpec((1, _BN, 1), lambda i: (1, i, 0))


def _gelu(v):
    return 0.5 * v * (1.0 + lax.erf(v * 0.7071067811865476))


def _layernorm(v, w, b):
    m = jnp.mean(v, axis=-1, keepdims=True)
    var = jnp.mean((v - m) ** 2, axis=-1, keepdims=True)
    return (v - m) * lax.rsqrt(var + 1e-5) * w + b


def _pack_bf16_pair(a, b):
    ba = lax.bitcast_convert_type(a, jnp.int32)
    bb = lax.bitcast_convert_type(b, jnp.int32)
    ra = ba + 32767 + (lax.shift_right_logical(ba, 16) & 1)
    rb = bb + 32767 + (lax.shift_right_logical(bb, 16) & 1)
    return lax.shift_right_logical(ra, 16) | (rb & jnp.int32(-65536))


def _tc_a_body(deg0, deg1, x, w1, w1a, w1b, g1_out, g1p_out, dinv_out):
    dinv = lax.rsqrt(deg0[0] + deg1[0] + 1.0)
    xv = x[...]
    g1_out[...] = dinv * jnp.dot(xv, w1[...], preferred_element_type=jnp.float32)
    pa = dinv * jnp.dot(xv, w1a[...], preferred_element_type=jnp.float32)
    pb = dinv * jnp.dot(xv, w1b[...], preferred_element_type=jnp.float32)
    g1p_out[...] = _pack_bf16_pair(pa, pb)
    dinv_out[...] = dinv


def _tc_a(deg3, x, w1, w1a, w1b):
    return pl.pallas_call(
        _tc_a_body,
        grid=(_GRID,),
        in_specs=[_d0_spec, _d1_spec, _row_spec, _mat_spec,
                  _hmat_spec, _hmat_spec],
        out_specs=[_row_spec, _half_spec, _col_spec],
        out_shape=[
            jax.ShapeDtypeStruct((N, D), jnp.float32),
            jax.ShapeDtypeStruct((N, DH), jnp.int32),
            jax.ShapeDtypeStruct((N, 1), jnp.float32),
        ],
    )(deg3, deg3, x, w1, w1a, w1b)


def _tc_b_body(s1, s1b, g1, dinv, b1, lnw, lnb, w2, w2a, w2b,
               g2_out, g2p_out):
    dv = dinv[...]
    v = dv * (s1[0] + s1b[0] + g1[...]) + b1[...]
    v = _gelu(_layernorm(v, lnw[...], lnb[...]))
    g2_out[...] = dv * jnp.dot(v, w2[...], preferred_element_type=jnp.float32)
    pa = dv * jnp.dot(v, w2a[...], preferred_element_type=jnp.float32)
    pb = dv * jnp.dot(v, w2b[...], preferred_element_type=jnp.float32)
    g2p_out[...] = _pack_bf16_pair(pa, pb)


def _tc_b(s1, g1, dinv, b1, lnw, lnb, w2, w2a, w2b):
    return pl.pallas_call(
        _tc_b_body,
        grid=(_GRID,),
        in_specs=[_s0_spec, _s1_spec, _row_spec, _col_spec,
                  _vec_spec, _vec_spec, _vec_spec, _mat_spec,
                  _hmat_spec, _hmat_spec],
        out_specs=[_row_spec, _half_spec],
        out_shape=[
            jax.ShapeDtypeStruct((N, D), jnp.float32),
            jax.ShapeDtypeStruct((N, DH), jnp.int32),
        ],
    )(s1, s1, g1, dinv, b1, lnw, lnb, w2, w2a, w2b)


def _tc_c_body(s2, s2b, g2, dinv, b2, lnw, lnb, x, out):
    v = dinv[...] * (s2[0] + s2b[0] + g2[...]) + b2[...]
    v = _layernorm(v, lnw[...], lnb[...]) + x[...]
    out[...] = _gelu(v)


def _tc_c(s2, g2, dinv, b2, lnw, lnb, x):
    return pl.pallas_call(
        _tc_c_body,
        grid=(_GRID,),
        in_specs=[_s0_spec, _s1_spec, _row_spec, _col_spec,
                  _vec_spec, _vec_spec, _vec_spec, _row_spec],
        out_specs=_row_spec,
        out_shape=jax.ShapeDtypeStruct((N, D), jnp.float32),
    )(s2, s2, g2, dinv, b2, lnw, lnb, x)


def kernel(x, edge_index, edge_attr, W1, b1, ln1_w, ln1_b, W2, b2, ln2_w, ln2_b):
    row = edge_index[0]
    col = edge_index[1]
    ew = edge_attr[:, 0]

    pad = EPAD - E
    zi = jnp.zeros((pad,), jnp.int32)
    rowp = jnp.concatenate([row, zi]).reshape(T, ENCH, ECH)
    colp = jnp.concatenate([col, zi]).reshape(T, ENCH, ECH)
    ewp = jnp.concatenate([ew, jnp.zeros((pad,), jnp.float32)]).reshape(
        T, ENCH, ECH)

    deg3 = _deg_kernel(colp, ewp).reshape(NC, NPAD, 1)

    g1, g1p, dinv = _tc_a(deg3, x, W1, W1[:, PA], W1[:, PB])

    s1 = _edge_kernel(rowp, colp, ewp, g1p)
    g2, g2p = _tc_b(s1, g1, dinv,
                    b1[None, :], ln1_w[None, :], ln1_b[None, :],
                    W2, W2[:, PA], W2[:, PB])

    s2 = _edge_kernel(rowp, colp, ewp, g2p)
    out = _tc_c(s2, g2, dinv,
                b2[None, :], ln2_w[None, :], ln2_b[None, :], x)
    return out
